# Initial kernel scaffold; baseline (speedup 1.0000x reference)
import jax
import jax.numpy as jnp
from jax.experimental import pallas as pl


def kernel(x, edge_index, edge_attr, entity_type, emb_table, lin1_W, lin1_b, ln_g, ln_b, l0_W, l0_b, out_W, out_b):
    raise NotImplementedError("write your pallas kernel here")


# speedup vs baseline: 2.3157x; 2.3157x over previous
import functools

import jax
import jax.numpy as jnp
from jax import lax
from jax.experimental import pallas as pl
from jax.experimental.pallas import tpu as pltpu
from jax.experimental.pallas import tpu_sc as plsc

N = 100000
NPAD = 102400
E = 3200000
DIN = 8
DH = 16
H = 64
NEMB = 4

NC = 2
NS = 16
NW = NC * NS

CH = 80
GB = 2000
RPI = GB // CH

EW = E // NW
ITG = EW // GB

GBS = 1600
RPS = GBS // CH
ET = E // NS
ITS = ET // GBS
NP = N // NS

_mesh = plsc.VectorSubcoreMesh(core_axis_name="c", subcore_axis_name="s")
_sc_params = pltpu.CompilerParams(use_tc_tiling_on_sc=False)


def _h_body(xp_ref, ohf_ref, p_ref, q_ref, h_ref):
    h_ref[...] = (
        jnp.dot(xp_ref[...], p_ref[...], preferred_element_type=jnp.float32)
        + jnp.dot(ohf_ref[...], q_ref[...], preferred_element_type=jnp.float32)
    )


_BH = 12500
_build_h = pl.pallas_call(
    _h_body,
    grid=(1,),
    in_specs=[
        pl.BlockSpec((_BH, H), lambda i: (0, 0)),
        pl.BlockSpec((_BH, 8 * NEMB), lambda i: (0, 0)),
        pl.BlockSpec((H, 128), lambda i: (0, 0)),
        pl.BlockSpec((8 * NEMB, 128), lambda i: (0, 0)),
    ],
    out_specs=pl.BlockSpec((_BH, 128), lambda i: (0, 0)),
    out_shape=jax.ShapeDtypeStruct((N // 8, 128), jnp.float32),
)


@functools.partial(
    pl.kernel,
    out_type=jax.ShapeDtypeStruct((E, DH), jnp.float32),
    mesh=_mesh,
    compiler_params=_sc_params,
    scratch_types=[
        pltpu.VMEM((GB,), jnp.int32),
        pltpu.VMEM((GB, DH), jnp.float32),
        pltpu.SemaphoreType.DMA,
    ],
)
def _sc_gather(h_hbm, src_hbm, xj_hbm, idx_v, rows_v, sem):
    wid = lax.axis_index("s") * NC + lax.axis_index("c")
    base = wid * EW

    def body(it, carry):
        e0 = pl.multiple_of(base + it * GB, 8)
        pltpu.sync_copy(src_hbm.at[pl.ds(e0, GB)], idx_v)

        def inner(j, carry2):
            o = pl.multiple_of(j * CH, 8)
            pltpu.async_copy(
                h_hbm.at[idx_v.at[pl.ds(o, CH)]],
                rows_v.at[pl.ds(o, CH)],
                sem,
            ).wait()
            return carry2

        lax.fori_loop(0, RPI, inner, 0)
        pltpu.sync_copy(rows_v, xj_hbm.at[pl.ds(e0, GB)])
        return carry

    lax.fori_loop(0, ITG, body, 0)


def _ln(m, g, b):
    mu = jnp.mean(m, axis=-1, keepdims=True)
    d = m - mu
    var = jnp.mean(d * d, axis=-1, keepdims=True)
    return d * lax.rsqrt(var + 1e-5) * g + b


_B8 = 1000
_BE = _B8 * 8


def _mlp_body(xj2_ref, ea_ref, whT_ref, weT_ref, b1_ref, g_ref, b_ref,
              w0T_ref, b0_ref, o_ref):
    msgs = []
    for j in range(8):
        xjj = xj2_ref[:, DH * j:DH * (j + 1)]
        eaj = ea_ref[:, 4 * j:4 * (j + 1)]
        m = (jnp.dot(xjj, whT_ref[...], preferred_element_type=jnp.float32)
             + jnp.dot(eaj, weT_ref[...], preferred_element_type=jnp.float32)
             + b1_ref[...])
        m = jnp.maximum(m, 0.0)
        m = _ln(m, g_ref[...], b_ref[...])
        m = (jnp.dot(m, w0T_ref[...], preferred_element_type=jnp.float32)
             + b0_ref[...])
        m = jnp.maximum(m, 0.0)
        msgs.append(_ln(m, g_ref[...], b_ref[...]))
    for c in range(4):
        o_ref[c] = jnp.concatenate(
            [msgs[j][:, c * DH:(c + 1) * DH] for j in range(8)], axis=-1)


_tc_mlp = pl.pallas_call(
    _mlp_body,
    grid=(E // _BE,),
    in_specs=[
        pl.BlockSpec((_B8, 128), lambda i: (i, 0)),
        pl.BlockSpec((_B8, 32), lambda i: (i, 0)),
        pl.BlockSpec((DH, H), lambda i: (0, 0)),
        pl.BlockSpec((4, H), lambda i: (0, 0)),
        pl.BlockSpec((1, H), lambda i: (0, 0)),
        pl.BlockSpec((1, H), lambda i: (0, 0)),
        pl.BlockSpec((1, H), lambda i: (0, 0)),
        pl.BlockSpec((H, H), lambda i: (0, 0)),
        pl.BlockSpec((1, H), lambda i: (0, 0)),
    ],
    out_specs=pl.BlockSpec((4, _B8, 128), lambda i: (0, i, 0)),
    out_shape=jax.ShapeDtypeStruct((4, E // 8, 128), jnp.float32),
)


@functools.partial(
    pl.kernel,
    out_type=jax.ShapeDtypeStruct((4, NPAD, DH), jnp.float32),
    mesh=_mesh,
    compiler_params=_sc_params,
    scratch_types=[
        pltpu.VMEM((RPS, CH), jnp.int32),
        pltpu.VMEM((GBS, DH), jnp.float32),
        pltpu.VMEM_SHARED((N, DH), jnp.float32),
    ],
)
def _sc_scatter(msg_hbm, dst2_hbm, z_hbm, agg_hbm, idx_v, upd_v, acc_sh):
    cid = lax.axis_index("c")
    sid = lax.axis_index("s")
    n0 = sid * NP
    ebase = sid * ET
    rbase = sid * (ET // CH)
    for cc in range(2):
        c = cid * 2 + cc
        pltpu.sync_copy(z_hbm.at[pl.ds(n0, NP)], acc_sh.at[pl.ds(n0, NP)])
        plsc.subcore_barrier()

        def body(it, carry):
            e0 = ebase + it * GBS
            r0 = rbase + it * RPS
            pltpu.sync_copy(dst2_hbm.at[pl.ds(r0, RPS)], idx_v)
            pltpu.sync_copy(msg_hbm.at[c, pl.ds(e0, GBS)], upd_v)

            def inner(j, carry2):
                o = pl.multiple_of(j * CH, 8)
                pltpu.sync_copy(
                    upd_v.at[pl.ds(o, CH)],
                    acc_sh.at[idx_v.at[j]],
                    add=True,
                )
                return carry2

            lax.fori_loop(0, RPS, inner, 0)
            return carry

        lax.fori_loop(0, ITS, body, 0)
        plsc.subcore_barrier()
        pltpu.sync_copy(acc_sh.at[pl.ds(n0, NP)], agg_hbm.at[c, pl.ds(n0, NP)])
        plsc.subcore_barrier()


_BA = 800
_BO = _BA * 8


def _out_body(a_ref, w_ref, b_ref, o_ref):
    outs = []
    for j in range(8):
        acc = b_ref[...]
        for c in range(4):
            acc = acc + jnp.dot(a_ref[c][:, DH * j:DH * (j + 1)],
                                w_ref[c * DH:(c + 1) * DH],
                                preferred_element_type=jnp.float32)
        outs.append(acc)
    o_ref[...] = jnp.concatenate(outs, axis=-1)


_tc_out = pl.pallas_call(
    _out_body,
    grid=(NPAD // _BO,),
    in_specs=[
        pl.BlockSpec((4, _BA, 128), lambda i: (0, i, 0)),
        pl.BlockSpec((H, H), lambda i: (0, 0)),
        pl.BlockSpec((1, H), lambda i: (0, 0)),
    ],
    out_specs=pl.BlockSpec((_BA, 8 * H), lambda i: (i, 0)),
    out_shape=jax.ShapeDtypeStruct((NPAD // 8, 8 * H), jnp.float32),
)


def kernel(x, edge_index, edge_attr, entity_type, emb_table, lin1_W, lin1_b,
           ln_g, ln_b, l0_W, l0_b, out_W, out_b):
    xp = x.reshape(N // 8, H)
    etp = entity_type.astype(jnp.int32).reshape(N // 8, 8)
    ohf = (etp[:, :, None] == jnp.arange(NEMB, dtype=jnp.int32)
           ).astype(jnp.float32).reshape(N // 8, 8 * NEMB)
    jj = jnp.arange(8)
    pmat = jnp.zeros((H, 128), jnp.float32)
    pmat = pmat.at[(8 * jj[:, None] + jnp.arange(8)[None, :]).reshape(-1),
                   (16 * jj[:, None] + jnp.arange(8)[None, :]).reshape(-1)
                   ].set(1.0)
    qmat = jnp.zeros((8 * NEMB, 128), jnp.float32)
    rows = (4 * jj[:, None, None] + jnp.arange(NEMB)[None, :, None]
            ) * jnp.ones((1, 1, DIN), jnp.int32)
    cols = (16 * jj[:, None, None] + 8
            + jnp.arange(DIN)[None, None, :]) * jnp.ones((1, NEMB, 1), jnp.int32)
    vals = jnp.broadcast_to(emb_table[None, :, :], (8, NEMB, DIN))
    qmat = qmat.at[rows.reshape(-1), cols.reshape(-1)].set(vals.reshape(-1))
    src = edge_index[0].astype(jnp.int32)
    dst2 = edge_index[1].astype(jnp.int32).reshape(E // CH, CH)
    whT = lin1_W[:, :DH].T
    weT = lin1_W[:, DH:].T
    b1 = lin1_b.reshape(1, H)
    g = ln_g.reshape(1, H)
    bb = ln_b.reshape(1, H)
    w0T = l0_W.T
    b0 = l0_b.reshape(1, H)
    woT = out_W.T
    bo = out_b.reshape(1, H)
    zeros = jnp.zeros((N, DH), jnp.float32)

    ea8 = edge_attr.reshape(E // 8, 32)

    h2 = _build_h(xp, ohf, pmat, qmat)
    xj = _sc_gather(h2.reshape(N, DH), src)
    m8 = _tc_mlp(xj.reshape(E // 8, 128), ea8, whT, weT, b1, g, bb,
                 w0T, b0)
    agg = _sc_scatter(m8.reshape(4, E, DH), dst2, zeros)
    out8 = _tc_out(agg.reshape(4, NPAD // 8, 128), woT, bo)
    return out8.reshape(NPAD, H)[:N]


# speedup vs baseline: 3.0560x; 1.3197x over previous
import functools

import jax
import jax.numpy as jnp
from jax import lax
from jax.experimental import pallas as pl
from jax.experimental.pallas import tpu as pltpu
from jax.experimental.pallas import tpu_sc as plsc

N = 100000
NPAD = 102400
E = 3200000
DIN = 8
DH = 16
H = 64
NEMB = 4

NC = 2
NS = 16
NW = NC * NS

CH = 80
GB = 2000
RPI = GB // CH

EW = E // NW
ITG = EW // GB

GBS = 1600
RPS = GBS // CH
ET = E // NS
ITS = ET // GBS
NP = N // NS

_mesh = plsc.VectorSubcoreMesh(core_axis_name="c", subcore_axis_name="s")
_sc_params = pltpu.CompilerParams(use_tc_tiling_on_sc=False)


def _h_body(xp_ref, ohf_ref, p_ref, q_ref, h_ref):
    h_ref[...] = (
        jnp.dot(xp_ref[...], p_ref[...], preferred_element_type=jnp.float32)
        + jnp.dot(ohf_ref[...], q_ref[...], preferred_element_type=jnp.float32)
    )


_BH = 12500
_build_h = pl.pallas_call(
    _h_body,
    grid=(1,),
    in_specs=[
        pl.BlockSpec((_BH, H), lambda i: (0, 0)),
        pl.BlockSpec((_BH, 8 * NEMB), lambda i: (0, 0)),
        pl.BlockSpec((H, 128), lambda i: (0, 0)),
        pl.BlockSpec((8 * NEMB, 128), lambda i: (0, 0)),
    ],
    out_specs=pl.BlockSpec((_BH, 128), lambda i: (0, 0)),
    out_shape=jax.ShapeDtypeStruct((N // 8, 128), jnp.float32),
)


@functools.partial(
    pl.kernel,
    out_type=jax.ShapeDtypeStruct((E, DH), jnp.float32),
    mesh=_mesh,
    compiler_params=_sc_params,
    scratch_types=[
        pltpu.VMEM((GB,), jnp.int32),
        pltpu.VMEM((GB, DH), jnp.float32),
        pltpu.SemaphoreType.DMA,
    ],
)
def _sc_gather(h_hbm, src_hbm, xj_hbm, idx_v, rows_v, sem):
    wid = lax.axis_index("s") * NC + lax.axis_index("c")
    base = wid * EW

    def body(it, carry):
        e0 = pl.multiple_of(base + it * GB, 8)
        pltpu.sync_copy(src_hbm.at[pl.ds(e0, GB)], idx_v)

        def inner(j, carry2):
            o = pl.multiple_of(j * CH, 8)
            pltpu.async_copy(
                h_hbm.at[idx_v.at[pl.ds(o, CH)]],
                rows_v.at[pl.ds(o, CH)],
                sem,
            ).wait()
            return carry2

        lax.fori_loop(0, RPI, inner, 0)
        pltpu.sync_copy(rows_v, xj_hbm.at[pl.ds(e0, GB)])
        return carry

    lax.fori_loop(0, ITG, body, 0)


def _ln(m, g, b):
    mu = jnp.mean(m, axis=-1, keepdims=True)
    d = m - mu
    var = jnp.mean(d * d, axis=-1, keepdims=True)
    return d * lax.rsqrt(var + 1e-5) * g + b


_B8 = 1600
_BE = _B8 * 8


def _mlp_body(xj2_ref, ea3_ref, whT_ref, weT_ref, b1_ref, g_ref, b_ref,
              w0T_ref, b0_ref, o_ref):
    msgs = []
    for j in range(8):
        xjj = xj2_ref[:, DH * j:DH * (j + 1)]
        m = (jnp.dot(xjj, whT_ref[...], preferred_element_type=jnp.float32)
             + b1_ref[...])
        for t in range(4):
            m = m + ea3_ref[t, :, j:j + 1] * weT_ref[t:t + 1, :]
        m = jnp.maximum(m, 0.0)
        m = _ln(m, g_ref[...], b_ref[...])
        m = (jnp.dot(m, w0T_ref[...], preferred_element_type=jnp.float32)
             + b0_ref[...])
        m = jnp.maximum(m, 0.0)
        msgs.append(_ln(m, g_ref[...], b_ref[...]))
    for c in range(4):
        o_ref[c] = jnp.concatenate(
            [msgs[j][:, c * DH:(c + 1) * DH] for j in range(8)], axis=-1)


_tc_mlp = pl.pallas_call(
    _mlp_body,
    grid=(E // _BE,),
    in_specs=[
        pl.BlockSpec((_B8, 128), lambda i: (i, 0)),
        pl.BlockSpec((4, _B8, 8), lambda i: (0, i, 0)),
        pl.BlockSpec((DH, H), lambda i: (0, 0)),
        pl.BlockSpec((4, H), lambda i: (0, 0)),
        pl.BlockSpec((1, H), lambda i: (0, 0)),
        pl.BlockSpec((1, H), lambda i: (0, 0)),
        pl.BlockSpec((1, H), lambda i: (0, 0)),
        pl.BlockSpec((H, H), lambda i: (0, 0)),
        pl.BlockSpec((1, H), lambda i: (0, 0)),
    ],
    out_specs=pl.BlockSpec((4, _B8, 128), lambda i: (0, i, 0)),
    out_shape=jax.ShapeDtypeStruct((4, E // 8, 128), jnp.float32),
)


@functools.partial(
    pl.kernel,
    out_type=jax.ShapeDtypeStruct((4, NPAD, DH), jnp.float32),
    mesh=_mesh,
    compiler_params=_sc_params,
    scratch_types=[
        pltpu.VMEM((RPS, CH), jnp.int32),
        pltpu.VMEM((GBS, DH), jnp.float32),
        pltpu.VMEM_SHARED((N, DH), jnp.float32),
    ],
)
def _sc_scatter(msg_hbm, dst2_hbm, z_hbm, agg_hbm, idx_v, upd_v, acc_sh):
    cid = lax.axis_index("c")
    sid = lax.axis_index("s")
    n0 = sid * NP
    ebase = sid * ET
    rbase = sid * (ET // CH)
    for cc in range(2):
        c = cid * 2 + cc
        pltpu.sync_copy(z_hbm.at[pl.ds(n0, NP)], acc_sh.at[pl.ds(n0, NP)])
        plsc.subcore_barrier()

        def body(it, carry):
            e0 = ebase + it * GBS
            r0 = rbase + it * RPS
            pltpu.sync_copy(dst2_hbm.at[pl.ds(r0, RPS)], idx_v)
            pltpu.sync_copy(msg_hbm.at[c, pl.ds(e0, GBS)], upd_v)

            def inner(j, carry2):
                o = pl.multiple_of(j * CH, 8)
                pltpu.sync_copy(
                    upd_v.at[pl.ds(o, CH)],
                    acc_sh.at[idx_v.at[j]],
                    add=True,
                )
                return carry2

            lax.fori_loop(0, RPS, inner, 0)
            return carry

        lax.fori_loop(0, ITS, body, 0)
        plsc.subcore_barrier()
        pltpu.sync_copy(acc_sh.at[pl.ds(n0, NP)], agg_hbm.at[c, pl.ds(n0, NP)])
        plsc.subcore_barrier()


_BA = 800
_BO = _BA * 8


def _out_body(a_ref, w_ref, b_ref, o_ref):
    outs = []
    for j in range(8):
        acc = b_ref[...]
        for c in range(4):
            acc = acc + jnp.dot(a_ref[c][:, DH * j:DH * (j + 1)],
                                w_ref[c * DH:(c + 1) * DH],
                                preferred_element_type=jnp.float32)
        outs.append(acc)
    o_ref[...] = jnp.concatenate(outs, axis=-1)


_tc_out = pl.pallas_call(
    _out_body,
    grid=(NPAD // _BO,),
    in_specs=[
        pl.BlockSpec((4, _BA, 128), lambda i: (0, i, 0)),
        pl.BlockSpec((H, H), lambda i: (0, 0)),
        pl.BlockSpec((1, H), lambda i: (0, 0)),
    ],
    out_specs=pl.BlockSpec((_BA, 8 * H), lambda i: (i, 0)),
    out_shape=jax.ShapeDtypeStruct((NPAD // 8, 8 * H), jnp.float32),
)


def kernel(x, edge_index, edge_attr, entity_type, emb_table, lin1_W, lin1_b,
           ln_g, ln_b, l0_W, l0_b, out_W, out_b):
    xp = x.reshape(N // 8, H)
    etp = entity_type.astype(jnp.int32).reshape(N // 8, 8)
    ohf = (etp[:, :, None] == jnp.arange(NEMB, dtype=jnp.int32)
           ).astype(jnp.float32).reshape(N // 8, 8 * NEMB)
    jj = jnp.arange(8)
    pmat = jnp.zeros((H, 128), jnp.float32)
    pmat = pmat.at[(8 * jj[:, None] + jnp.arange(8)[None, :]).reshape(-1),
                   (16 * jj[:, None] + jnp.arange(8)[None, :]).reshape(-1)
                   ].set(1.0)
    qmat = jnp.zeros((8 * NEMB, 128), jnp.float32)
    rows = (4 * jj[:, None, None] + jnp.arange(NEMB)[None, :, None]
            ) * jnp.ones((1, 1, DIN), jnp.int32)
    cols = (16 * jj[:, None, None] + 8
            + jnp.arange(DIN)[None, None, :]) * jnp.ones((1, NEMB, 1), jnp.int32)
    vals = jnp.broadcast_to(emb_table[None, :, :], (8, NEMB, DIN))
    qmat = qmat.at[rows.reshape(-1), cols.reshape(-1)].set(vals.reshape(-1))
    src = edge_index[0].astype(jnp.int32)
    dst2 = edge_index[1].astype(jnp.int32).reshape(E // CH, CH)
    whT = lin1_W[:, :DH].T
    weT = lin1_W[:, DH:].T
    b1 = lin1_b.reshape(1, H)
    g = ln_g.reshape(1, H)
    bb = ln_b.reshape(1, H)
    w0T = l0_W.T
    b0 = l0_b.reshape(1, H)
    woT = out_W.T
    bo = out_b.reshape(1, H)
    zeros = jnp.zeros((N, DH), jnp.float32)

    ea3 = edge_attr.T.reshape(4, E // 8, 8)

    h2 = _build_h(xp, ohf, pmat, qmat)
    xj = _sc_gather(h2.reshape(N, DH), src)
    m8 = _tc_mlp(xj.reshape(E // 8, 128), ea3, whT, weT, b1, g, bb,
                 w0T, b0)
    agg = _sc_scatter(m8.reshape(4, E, DH), dst2, zeros)
    out8 = _tc_out(agg.reshape(4, NPAD // 8, 128), woT, bo)
    return out8.reshape(NPAD, H)[:N]
